# Initial kernel scaffold; baseline (speedup 1.0000x reference)
import functools
import jax
import jax.numpy as jnp
from jax import lax
from jax.experimental import pallas as pl
from jax.experimental.pallas import tpu as pltpu
from jax.experimental.pallas import tpu_sc as plsc

N = 10000
E = 320000
D = 64
NC = 2
NS = 16
NW = NC * NS
CH = 128
NCHUNK = E // CH
T_FULL = NCHUNK // NW
T_REM = NCHUNK - T_FULL * NW
RPT = N // NS
RCP = 125

BN = 1000

_mesh = lambda: plsc.VectorSubcoreMesh(core_axis_name="c", subcore_axis_name="s")


def _sc_deg(dst):
    @functools.partial(
        pl.kernel,
        out_type=jax.ShapeDtypeStruct((NW, N), jnp.float32),
        mesh=_mesh(),
        scratch_types=[
            pltpu.VMEM((N,), jnp.float32),
            pltpu.VMEM((CH,), jnp.int32),
        ],
    )
    def k(dst_hbm, pdeg_hbm, deg_v, idx_v):
        c = lax.axis_index("c")
        s = lax.axis_index("s")
        wid = s * NC + c
        zeros = jnp.zeros((16,), jnp.float32)
        ones = jnp.ones((16,), jnp.float32)

        def zero_body(i, carry):
            deg_v[pl.ds(i * 16, 16)] = zeros
            return carry

        lax.fori_loop(0, N // 16, zero_body, 0)

        def do_chunk(t):
            base = (wid + NW * t) * CH
            pltpu.sync_copy(dst_hbm.at[pl.ds(base, CH)], idx_v)
            for j in range(CH // 16):
                idx = idx_v[pl.ds(j * 16, 16)]
                plsc.addupdate_scatter(deg_v, [idx], ones)

        def t_body(t, carry):
            do_chunk(t)
            return carry

        lax.fori_loop(0, T_FULL, t_body, 0)

        @pl.when(wid < T_REM)
        def _():
            do_chunk(T_FULL)

        pltpu.sync_copy(deg_v, pdeg_hbm.at[wid])

    return k(dst)


def _sc_scatter(hwp, src, dst):
    @functools.partial(
        pl.kernel,
        out_type=jax.ShapeDtypeStruct((NC * N, D), jnp.float32),
        mesh=_mesh(),
        scratch_types=[
            pltpu.VMEM((CH,), jnp.int32),
            pltpu.VMEM((CH,), jnp.int32),
            pltpu.VMEM((CH, D), jnp.float32),
            pltpu.VMEM_SHARED((N, D), jnp.float32),
            pltpu.SemaphoreType.DMA,
        ],
    )
    def k(hwp_hbm, src_hbm, dst_hbm, out_hbm, sidx_v, didx_v, rows_v, acc_sh, sem):
        c = lax.axis_index("c")
        s = lax.axis_index("s")
        wid = s * NC + c
        zeros = jnp.zeros((16,), jnp.float32)

        def zero_body(i, carry):
            for j in range(D // 16):
                rows_v[i, pl.ds(j * 16, 16)] = zeros
            return carry

        lax.fori_loop(0, CH, zero_body, 0)
        for j in range(RPT // RCP):
            pltpu.sync_copy(rows_v.at[pl.ds(0, RCP)],
                            acc_sh.at[pl.ds(s * RPT + j * RCP, RCP)])
        plsc.subcore_barrier()

        def do_chunk(t):
            base = (wid + NW * t) * CH
            pltpu.sync_copy(src_hbm.at[pl.ds(base, CH)], sidx_v)
            pltpu.sync_copy(dst_hbm.at[pl.ds(base, CH)], didx_v)
            pltpu.async_copy(hwp_hbm.at[sidx_v], rows_v, sem).wait()
            pltpu.sync_copy(rows_v, acc_sh.at[didx_v], add=True)

        def t_body(t, carry):
            do_chunk(t)
            return carry

        lax.fori_loop(0, T_FULL, t_body, 0)

        @pl.when(wid < T_REM)
        def _():
            do_chunk(T_FULL)

        plsc.subcore_barrier()

        for j in range(RPT // RCP):
            r0 = s * RPT + j * RCP
            pltpu.sync_copy(acc_sh.at[pl.ds(r0, RCP)], rows_v.at[pl.ds(0, RCP)])
            pltpu.sync_copy(rows_v.at[pl.ds(0, RCP)], out_hbm.at[pl.ds(c * N + r0, RCP)])

    return k(hwp, src, dst)


def _lrelu(v):
    return jnp.where(v >= 0, v, 0.01 * v)


def _dinv_of(pdeg_blk):
    deg = jnp.sum(pdeg_blk, axis=0) + 1.0
    return lax.rsqrt(deg)[:, None]


def _dot(a, b):
    return jnp.dot(a, b, preferred_element_type=jnp.float32)


def _tc_encoder(x, W1, b1, W2, b2, g1W, pdeg):
    def body(x_r, w1_r, b1_r, w2_r, b2_r, g1w_r, pdeg_r, out_r):
        h = _lrelu(_dot(x_r[...], w1_r[...]) + b1_r[...])
        h = _lrelu(_dot(h, w2_r[...]) + b2_r[...])
        out_r[...] = _dot(h, g1w_r[...]) * _dinv_of(pdeg_r[...])

    full = lambda shape: pl.BlockSpec(shape, lambda i: (0,) * len(shape))
    return pl.pallas_call(
        body,
        grid=(N // BN,),
        in_specs=[
            pl.BlockSpec((BN, 128), lambda i: (i, 0)),
            full((128, 128)),
            full((1, 128)),
            full((128, D)),
            full((1, D)),
            full((D, D)),
            pl.BlockSpec((NW, BN), lambda i: (0, i)),
        ],
        out_specs=pl.BlockSpec((BN, D), lambda i: (i, 0)),
        out_shape=jax.ShapeDtypeStruct((N, D), jnp.float32),
    )(x, W1, b1, W2, b2, g1W, pdeg)


def _tc_combine(p0, p1, hwp, pdeg, gW, gb):
    def body(p0_r, p1_r, hwp_r, pdeg_r, gw_r, gb_r, out_r):
        dinv = _dinv_of(pdeg_r[...])
        h = _lrelu((p0_r[...] + p1_r[...] + hwp_r[...]) * dinv + gb_r[...])
        out_r[...] = _dot(h, gw_r[...]) * dinv

    full = lambda shape: pl.BlockSpec(shape, lambda i: (0,) * len(shape))
    blk = pl.BlockSpec((BN, D), lambda i: (i, 0))
    return pl.pallas_call(
        body,
        grid=(N // BN,),
        in_specs=[blk, blk, blk,
                  pl.BlockSpec((NW, BN), lambda i: (0, i)),
                  full((D, D)), full((1, D))],
        out_specs=pl.BlockSpec((BN, D), lambda i: (i, 0)),
        out_shape=jax.ShapeDtypeStruct((N, D), jnp.float32),
    )(p0, p1, hwp, pdeg, gW, gb)


def _tc_final(p0, p1, hwp, pdeg, gb, cW, cb):
    def body(p0_r, p1_r, hwp_r, pdeg_r, gb_r, cw_r, cb_r, h_r, log_r):
        dinv = _dinv_of(pdeg_r[...])
        h = _lrelu((p0_r[...] + p1_r[...] + hwp_r[...]) * dinv + gb_r[...])
        h_r[...] = h
        log_r[...] = _dot(h, cw_r[...]) + cb_r[...]

    full = lambda shape: pl.BlockSpec(shape, lambda i: (0,) * len(shape))
    blk = pl.BlockSpec((BN, D), lambda i: (i, 0))
    return pl.pallas_call(
        body,
        grid=(N // BN,),
        in_specs=[blk, blk, blk,
                  pl.BlockSpec((NW, BN), lambda i: (0, i)),
                  full((1, D)), full((D, 2)), full((1, 2))],
        out_specs=[pl.BlockSpec((BN, D), lambda i: (i, 0)),
                   pl.BlockSpec((BN, 2), lambda i: (i, 0))],
        out_shape=[jax.ShapeDtypeStruct((N, D), jnp.float32),
                   jax.ShapeDtypeStruct((N, 2), jnp.float32)],
    )(p0, p1, hwp, pdeg, gb, cW, cb)


def kernel(x, edge_index, W1, b1, W2, b2, g1W, g1b, g2W, g2b, cW, cb):
    src = edge_index[0]
    dst = edge_index[1]
    pdeg = _sc_deg(dst)
    hw1p = _tc_encoder(x, W1, b1.reshape(1, -1), W2, b2.reshape(1, -1), g1W, pdeg)
    part1 = _sc_scatter(hw1p, src, dst)
    hw2p = _tc_combine(part1[:N], part1[N:], hw1p, pdeg, g2W, g1b.reshape(1, -1))
    part2 = _sc_scatter(hw2p, src, dst)
    h, logits = _tc_final(part2[:N], part2[N:], hw2p, pdeg, g2b.reshape(1, -1),
                          cW, cb.reshape(1, -1))
    return (logits, h)


# speedup vs baseline: 18.4879x; 18.4879x over previous
import functools
import jax
import jax.numpy as jnp
from jax import lax
from jax.experimental import pallas as pl
from jax.experimental.pallas import tpu as pltpu
from jax.experimental.pallas import tpu_sc as plsc

N = 10000
E = 320000
D = 64
NC = 2
NS = 16
NW = NC * NS
CH = 128
NCHUNK = E // CH
T_FULL = NCHUNK // NW
T_REM = NCHUNK - T_FULL * NW
RPT = N // NS
RCP = 125

BN = 1024

_mesh = lambda: plsc.VectorSubcoreMesh(core_axis_name="c", subcore_axis_name="s")


def _sc_deg(dst):
    @functools.partial(
        pl.kernel,
        out_type=jax.ShapeDtypeStruct((NW, N), jnp.float32),
        mesh=_mesh(),
        compiler_params=pltpu.CompilerParams(needs_layout_passes=False, use_tc_tiling_on_sc=False),
        scratch_types=[
            pltpu.VMEM((N,), jnp.float32),
            pltpu.VMEM((CH,), jnp.int32),
        ],
    )
    def k(dst_hbm, pdeg_hbm, deg_v, idx_v):
        c = lax.axis_index("c")
        s = lax.axis_index("s")
        wid = s * NC + c
        zeros = jnp.zeros((16,), jnp.float32)
        ones = jnp.ones((16,), jnp.float32)

        def zero_body(i, carry):
            deg_v[pl.ds(i * 16, 16)] = zeros
            return carry

        lax.fori_loop(0, N // 16, zero_body, 0)

        def do_chunk(t):
            base = (wid + NW * t) * CH
            pltpu.sync_copy(dst_hbm.at[pl.ds(base, CH)], idx_v)
            for j in range(CH // 16):
                idx = idx_v[pl.ds(j * 16, 16)]
                plsc.addupdate_scatter(deg_v, [idx], ones)

        def t_body(t, carry):
            do_chunk(t)
            return carry

        lax.fori_loop(0, T_FULL, t_body, 0)

        @pl.when(wid < T_REM)
        def _():
            do_chunk(T_FULL)

        pltpu.sync_copy(deg_v, pdeg_hbm.at[wid])

    return k(dst)


def _sc_scatter(hwp, src, dst):
    @functools.partial(
        pl.kernel,
        out_type=jax.ShapeDtypeStruct((NC * N, D), jnp.float32),
        mesh=_mesh(),
        compiler_params=pltpu.CompilerParams(needs_layout_passes=False, use_tc_tiling_on_sc=False),
        scratch_types=[
            pltpu.VMEM((CH,), jnp.int32),
            pltpu.VMEM((CH,), jnp.int32),
            pltpu.VMEM((CH, D), jnp.float32),
            pltpu.VMEM_SHARED((N, D), jnp.float32),
            pltpu.SemaphoreType.DMA,
        ],
    )
    def k(hwp_hbm, src_hbm, dst_hbm, out_hbm, sidx_v, didx_v, rows_v, acc_sh, sem):
        c = lax.axis_index("c")
        s = lax.axis_index("s")
        wid = s * NC + c
        zeros = jnp.zeros((16,), jnp.float32)

        def zero_body(i, carry):
            for j in range(D // 16):
                rows_v[i, pl.ds(j * 16, 16)] = zeros
            return carry

        lax.fori_loop(0, CH, zero_body, 0)
        for t in range(5):
            cid = s + NS * t
            base = pl.multiple_of(cid * CH, CH)

            @pl.when(cid < N // CH)
            def _():
                pltpu.sync_copy(rows_v, acc_sh.at[pl.ds(base, CH)])

            @pl.when(cid == N // CH)
            def _():
                pltpu.sync_copy(rows_v.at[pl.ds(0, N % CH)],
                                acc_sh.at[pl.ds((N // CH) * CH, N % CH)])

        plsc.subcore_barrier()

        def do_chunk(t):
            base = (wid + NW * t) * CH
            pltpu.sync_copy(src_hbm.at[pl.ds(base, CH)], sidx_v)
            pltpu.sync_copy(dst_hbm.at[pl.ds(base, CH)], didx_v)
            pltpu.async_copy(hwp_hbm.at[sidx_v], rows_v, sem).wait()
            pltpu.sync_copy(rows_v, acc_sh.at[didx_v], add=True)

        def t_body(t, carry):
            do_chunk(t)
            return carry

        lax.fori_loop(0, T_FULL, t_body, 0)

        @pl.when(wid < T_REM)
        def _():
            do_chunk(T_FULL)

        plsc.subcore_barrier()

        for t in range(5):
            cid = s + NS * t
            base = pl.multiple_of(cid * CH, CH)

            @pl.when(cid < N // CH)
            def _():
                pltpu.sync_copy(acc_sh.at[pl.ds(base, CH)], rows_v)
                pltpu.sync_copy(rows_v, out_hbm.at[pl.ds(c * N + base, CH)])

            @pl.when(cid == N // CH)
            def _():
                tail = N % CH
                t0 = (N // CH) * CH
                pltpu.sync_copy(acc_sh.at[pl.ds(t0, tail)], rows_v.at[pl.ds(0, tail)])
                pltpu.sync_copy(rows_v.at[pl.ds(0, tail)],
                                out_hbm.at[pl.ds(c * N + t0, tail)])

    return k(hwp, src, dst)


def _lrelu(v):
    return jnp.where(v >= 0, v, 0.01 * v)


def _dinv_of(pdeg_blk):
    deg = jnp.sum(pdeg_blk, axis=0) + 1.0
    return lax.rsqrt(deg)[:, None]


def _dot(a, b):
    return jnp.dot(a, b, preferred_element_type=jnp.float32)


def _tc_encoder(x, W1, b1, W2, b2, g1W, pdeg):
    def body(x_r, w1_r, b1_r, w2_r, b2_r, g1w_r, pdeg_r, out_r):
        h = _lrelu(_dot(x_r[...], w1_r[...]) + b1_r[...])
        h = _lrelu(_dot(h, w2_r[...]) + b2_r[...])
        out_r[...] = _dot(h, g1w_r[...]) * _dinv_of(pdeg_r[...])

    full = lambda shape: pl.BlockSpec(shape, lambda i: (0,) * len(shape))
    return pl.pallas_call(
        body,
        grid=(pl.cdiv(N, BN),),
        in_specs=[
            pl.BlockSpec((BN, 128), lambda i: (i, 0)),
            full((128, 128)),
            full((1, 128)),
            full((128, D)),
            full((1, D)),
            full((D, D)),
            pl.BlockSpec((NW, BN), lambda i: (0, i)),
        ],
        out_specs=pl.BlockSpec((BN, D), lambda i: (i, 0)),
        out_shape=jax.ShapeDtypeStruct((N, D), jnp.float32),
    )(x, W1, b1, W2, b2, g1W, pdeg)


def _tc_combine(p0, p1, hwp, pdeg, gW, gb):
    def body(p0_r, p1_r, hwp_r, pdeg_r, gw_r, gb_r, out_r):
        dinv = _dinv_of(pdeg_r[...])
        h = _lrelu((p0_r[...] + p1_r[...] + hwp_r[...]) * dinv + gb_r[...])
        out_r[...] = _dot(h, gw_r[...]) * dinv

    full = lambda shape: pl.BlockSpec(shape, lambda i: (0,) * len(shape))
    blk = pl.BlockSpec((BN, D), lambda i: (i, 0))
    return pl.pallas_call(
        body,
        grid=(pl.cdiv(N, BN),),
        in_specs=[blk, blk, blk,
                  pl.BlockSpec((NW, BN), lambda i: (0, i)),
                  full((D, D)), full((1, D))],
        out_specs=pl.BlockSpec((BN, D), lambda i: (i, 0)),
        out_shape=jax.ShapeDtypeStruct((N, D), jnp.float32),
    )(p0, p1, hwp, pdeg, gW, gb)


def _tc_final(p0, p1, hwp, pdeg, gb, cW, cb):
    def body(p0_r, p1_r, hwp_r, pdeg_r, gb_r, cw_r, cb_r, h_r, log_r):
        dinv = _dinv_of(pdeg_r[...])
        h = _lrelu((p0_r[...] + p1_r[...] + hwp_r[...]) * dinv + gb_r[...])
        h_r[...] = h
        log_r[...] = _dot(h, cw_r[...]) + cb_r[...]

    full = lambda shape: pl.BlockSpec(shape, lambda i: (0,) * len(shape))
    blk = pl.BlockSpec((BN, D), lambda i: (i, 0))
    return pl.pallas_call(
        body,
        grid=(pl.cdiv(N, BN),),
        in_specs=[blk, blk, blk,
                  pl.BlockSpec((NW, BN), lambda i: (0, i)),
                  full((1, D)), full((D, 2)), full((1, 2))],
        out_specs=[pl.BlockSpec((BN, D), lambda i: (i, 0)),
                   pl.BlockSpec((BN, 2), lambda i: (i, 0))],
        out_shape=[jax.ShapeDtypeStruct((N, D), jnp.float32),
                   jax.ShapeDtypeStruct((N, 2), jnp.float32)],
    )(p0, p1, hwp, pdeg, gb, cW, cb)


def kernel(x, edge_index, W1, b1, W2, b2, g1W, g1b, g2W, g2b, cW, cb):
    src = edge_index[0]
    dst = edge_index[1]
    pdeg = _sc_deg(dst)
    hw1p = _tc_encoder(x, W1, b1.reshape(1, -1), W2, b2.reshape(1, -1), g1W, pdeg)
    part1 = _sc_scatter(hw1p, src, dst)
    hw2p = _tc_combine(part1[:N], part1[N:], hw1p, pdeg, g2W, g1b.reshape(1, -1))
    part2 = _sc_scatter(hw2p, src, dst)
    h, logits = _tc_final(part2[:N], part2[N:], hw2p, pdeg, g2b.reshape(1, -1),
                          cW, cb.reshape(1, -1))
    return (logits, h)


# speedup vs baseline: 34.6028x; 1.8716x over previous
import functools
import jax
import jax.numpy as jnp
from jax import lax
from jax.experimental import pallas as pl
from jax.experimental.pallas import tpu as pltpu
from jax.experimental.pallas import tpu_sc as plsc

N = 10000
E = 320000
D = 64
NC = 2
NS = 16
NW = NC * NS
CH = 128
EPW = E // NW
NFULL = EPW // CH
TAIL = EPW - NFULL * CH
NBUF = 6
NGRP = NFULL // NBUF

BN = 1024

_mesh = lambda: plsc.VectorSubcoreMesh(core_axis_name="c", subcore_axis_name="s")


def _sc_deg(dst):
    @functools.partial(
        pl.kernel,
        out_type=jax.ShapeDtypeStruct((NW, N), jnp.float32),
        mesh=_mesh(),
        compiler_params=pltpu.CompilerParams(needs_layout_passes=False, use_tc_tiling_on_sc=False),
        scratch_types=[
            pltpu.VMEM((N,), jnp.float32),
            pltpu.VMEM((EPW,), jnp.int32),
        ],
    )
    def k(dst_hbm, pdeg_hbm, deg_v, idx_v):
        c = lax.axis_index("c")
        s = lax.axis_index("s")
        wid = s * NC + c
        zeros = jnp.zeros((16,), jnp.float32)
        ones = jnp.ones((16,), jnp.float32)

        def zero_body(i, carry):
            deg_v[pl.ds(i * 16, 16)] = zeros
            return carry

        lax.fori_loop(0, N // 16, zero_body, 0)

        pltpu.sync_copy(dst_hbm.at[pl.ds(wid * EPW, EPW)], idx_v)

        def h_body(i, carry):
            for j in range(25):
                idx = idx_v[pl.ds((i * 25 + j) * 16, 16)]
                plsc.addupdate_scatter(deg_v, [idx], ones)
            return carry

        lax.fori_loop(0, EPW // (25 * 16), h_body, 0)

        pltpu.sync_copy(deg_v, pdeg_hbm.at[wid])

    return k(dst)


def _sc_scatter(hwp, src, dst):
    @functools.partial(
        pl.kernel,
        out_type=jax.ShapeDtypeStruct((NC * N, D), jnp.float32),
        mesh=_mesh(),
        compiler_params=pltpu.CompilerParams(needs_layout_passes=False, use_tc_tiling_on_sc=False),
        scratch_types=[
            pltpu.VMEM((EPW,), jnp.int32),
            pltpu.VMEM((EPW,), jnp.int32),
            pltpu.VMEM((NBUF, CH, D), jnp.float32),
            pltpu.VMEM_SHARED((N, D), jnp.float32),
            pltpu.SemaphoreType.DMA,
        ],
    )
    def k(hwp_hbm, src_hbm, dst_hbm, out_hbm, sidx_v, didx_v, rows_v, acc_sh, sem):
        c = lax.axis_index("c")
        s = lax.axis_index("s")
        wid = s * NC + c
        zeros = jnp.zeros((16,), jnp.float32)

        def zero_body(i, carry):
            for j in range(D // 16):
                rows_v[0, i, pl.ds(j * 16, 16)] = zeros
            return carry

        lax.fori_loop(0, CH, zero_body, 0)
        for t in range(5):
            cid = s + NS * t
            base = pl.multiple_of(cid * CH, CH)

            @pl.when(cid < N // CH)
            def _():
                pltpu.sync_copy(rows_v.at[0], acc_sh.at[pl.ds(base, CH)])

            @pl.when(cid == N // CH)
            def _():
                pltpu.sync_copy(rows_v.at[0].at[pl.ds(0, N % CH)],
                                acc_sh.at[pl.ds((N // CH) * CH, N % CH)])

        ebase = wid * EPW
        pltpu.sync_copy(src_hbm.at[pl.ds(ebase, EPW)], sidx_v)
        pltpu.sync_copy(dst_hbm.at[pl.ds(ebase, EPW)], didx_v)
        plsc.subcore_barrier()

        def g_body(g, carry):
            gb = g * (NBUF * CH)
            cps = [
                pltpu.async_copy(
                    hwp_hbm.at[sidx_v.at[pl.ds(gb + b * CH, CH)]],
                    rows_v.at[b], sem)
                for b in range(NBUF)
            ]
            for cp in cps:
                cp.wait()
            for b in range(NBUF):
                pltpu.sync_copy(rows_v.at[b],
                                acc_sh.at[didx_v.at[pl.ds(gb + b * CH, CH)]],
                                add=True)
            return carry

        lax.fori_loop(0, NGRP, g_body, 0)

        tb = NFULL * CH
        pltpu.async_copy(hwp_hbm.at[sidx_v.at[pl.ds(tb, TAIL)]],
                         rows_v.at[0].at[pl.ds(0, TAIL)], sem).wait()
        pltpu.sync_copy(rows_v.at[0].at[pl.ds(0, TAIL)],
                        acc_sh.at[didx_v.at[pl.ds(tb, TAIL)]], add=True)

        plsc.subcore_barrier()

        for t in range(5):
            cid = s + NS * t
            base = pl.multiple_of(cid * CH, CH)

            @pl.when(cid < N // CH)
            def _():
                pltpu.sync_copy(acc_sh.at[pl.ds(base, CH)], rows_v.at[0])
                pltpu.sync_copy(rows_v.at[0], out_hbm.at[pl.ds(c * N + base, CH)])

            @pl.when(cid == N // CH)
            def _():
                tail = N % CH
                t0 = (N // CH) * CH
                pltpu.sync_copy(acc_sh.at[pl.ds(t0, tail)],
                                rows_v.at[0].at[pl.ds(0, tail)])
                pltpu.sync_copy(rows_v.at[0].at[pl.ds(0, tail)],
                                out_hbm.at[pl.ds(c * N + t0, tail)])

    return k(hwp, src, dst)


def _lrelu(v):
    return jnp.where(v >= 0, v, 0.01 * v)


def _dinv_of(pdeg_blk):
    deg = jnp.sum(pdeg_blk, axis=0) + 1.0
    return lax.rsqrt(deg)[:, None]


def _dot(a, b):
    return jnp.dot(a, b, preferred_element_type=jnp.float32)


def _tc_encoder(x, W1, b1, W2, b2, g1W, pdeg):
    def body(x_r, w1_r, b1_r, w2_r, b2_r, g1w_r, pdeg_r, out_r):
        h = _lrelu(_dot(x_r[...], w1_r[...]) + b1_r[...])
        h = _lrelu(_dot(h, w2_r[...]) + b2_r[...])
        out_r[...] = _dot(h, g1w_r[...]) * _dinv_of(pdeg_r[...])

    full = lambda shape: pl.BlockSpec(shape, lambda i: (0,) * len(shape))
    return pl.pallas_call(
        body,
        grid=(pl.cdiv(N, BN),),
        in_specs=[
            pl.BlockSpec((BN, 128), lambda i: (i, 0)),
            full((128, 128)),
            full((1, 128)),
            full((128, D)),
            full((1, D)),
            full((D, D)),
            pl.BlockSpec((NW, BN), lambda i: (0, i)),
        ],
        out_specs=pl.BlockSpec((BN, D), lambda i: (i, 0)),
        out_shape=jax.ShapeDtypeStruct((N, D), jnp.float32),
    )(x, W1, b1, W2, b2, g1W, pdeg)


def _tc_combine(p0, p1, hwp, pdeg, gW, gb):
    def body(p0_r, p1_r, hwp_r, pdeg_r, gw_r, gb_r, out_r):
        dinv = _dinv_of(pdeg_r[...])
        h = _lrelu((p0_r[...] + p1_r[...] + hwp_r[...]) * dinv + gb_r[...])
        out_r[...] = _dot(h, gw_r[...]) * dinv

    full = lambda shape: pl.BlockSpec(shape, lambda i: (0,) * len(shape))
    blk = pl.BlockSpec((BN, D), lambda i: (i, 0))
    return pl.pallas_call(
        body,
        grid=(pl.cdiv(N, BN),),
        in_specs=[blk, blk, blk,
                  pl.BlockSpec((NW, BN), lambda i: (0, i)),
                  full((D, D)), full((1, D))],
        out_specs=pl.BlockSpec((BN, D), lambda i: (i, 0)),
        out_shape=jax.ShapeDtypeStruct((N, D), jnp.float32),
    )(p0, p1, hwp, pdeg, gW, gb)


def _tc_final(p0, p1, hwp, pdeg, gb, cW, cb):
    def body(p0_r, p1_r, hwp_r, pdeg_r, gb_r, cw_r, cb_r, h_r, log_r):
        dinv = _dinv_of(pdeg_r[...])
        h = _lrelu((p0_r[...] + p1_r[...] + hwp_r[...]) * dinv + gb_r[...])
        h_r[...] = h
        log_r[...] = _dot(h, cw_r[...]) + cb_r[...]

    full = lambda shape: pl.BlockSpec(shape, lambda i: (0,) * len(shape))
    blk = pl.BlockSpec((BN, D), lambda i: (i, 0))
    return pl.pallas_call(
        body,
        grid=(pl.cdiv(N, BN),),
        in_specs=[blk, blk, blk,
                  pl.BlockSpec((NW, BN), lambda i: (0, i)),
                  full((1, D)), full((D, 2)), full((1, 2))],
        out_specs=[pl.BlockSpec((BN, D), lambda i: (i, 0)),
                   pl.BlockSpec((BN, 2), lambda i: (i, 0))],
        out_shape=[jax.ShapeDtypeStruct((N, D), jnp.float32),
                   jax.ShapeDtypeStruct((N, 2), jnp.float32)],
    )(p0, p1, hwp, pdeg, gb, cW, cb)


def kernel(x, edge_index, W1, b1, W2, b2, g1W, g1b, g2W, g2b, cW, cb):
    src = edge_index[0]
    dst = edge_index[1]
    pdeg = _sc_deg(dst)
    hw1p = _tc_encoder(x, W1, b1.reshape(1, -1), W2, b2.reshape(1, -1), g1W, pdeg)
    part1 = _sc_scatter(hw1p, src, dst)
    hw2p = _tc_combine(part1[:N], part1[N:], hw1p, pdeg, g2W, g1b.reshape(1, -1))
    part2 = _sc_scatter(hw2p, src, dst)
    h, logits = _tc_final(part2[:N], part2[N:], hw2p, pdeg, g2b.reshape(1, -1),
                          cW, cb.reshape(1, -1))
    return (logits, h)


# speedup vs baseline: 39.2488x; 1.1343x over previous
import functools
import jax
import jax.numpy as jnp
from jax import lax
from jax.experimental import pallas as pl
from jax.experimental.pallas import tpu as pltpu
from jax.experimental.pallas import tpu_sc as plsc

N = 10000
E = 320000
D = 64
NC = 2
NS = 16
NW = NC * NS
CH = 128
EPW = E // NW
NFULL = EPW // CH
TAIL = EPW - NFULL * CH
NB = 3
NPAIR = NFULL // (2 * NB)

BN = 1024

_mesh = lambda: plsc.VectorSubcoreMesh(core_axis_name="c", subcore_axis_name="s")


def _sc_deg(dst):
    @functools.partial(
        pl.kernel,
        out_type=jax.ShapeDtypeStruct((NW, N), jnp.float32),
        mesh=_mesh(),
        compiler_params=pltpu.CompilerParams(needs_layout_passes=False, use_tc_tiling_on_sc=False),
        scratch_types=[
            pltpu.VMEM((N,), jnp.float32),
            pltpu.VMEM((EPW,), jnp.int32),
        ],
    )
    def k(dst_hbm, pdeg_hbm, deg_v, idx_v):
        c = lax.axis_index("c")
        s = lax.axis_index("s")
        wid = s * NC + c
        zeros = jnp.zeros((16,), jnp.float32)
        ones = jnp.ones((16,), jnp.float32)

        def zero_body(i, carry):
            deg_v[pl.ds(i * 16, 16)] = zeros
            return carry

        lax.fori_loop(0, N // 16, zero_body, 0)

        pltpu.sync_copy(dst_hbm.at[pl.ds(wid * EPW, EPW)], idx_v)

        def h_body(i, carry):
            for j in range(25):
                idx = idx_v[pl.ds((i * 25 + j) * 16, 16)]
                plsc.addupdate_scatter(deg_v, [idx], ones)
            return carry

        lax.fori_loop(0, EPW // (25 * 16), h_body, 0)

        pltpu.sync_copy(deg_v, pdeg_hbm.at[wid])

    return k(dst)


def _sc_scatter(hwp, src, dst):
    @functools.partial(
        pl.kernel,
        out_type=jax.ShapeDtypeStruct((NC * N, D), jnp.float32),
        mesh=_mesh(),
        compiler_params=pltpu.CompilerParams(needs_layout_passes=False, use_tc_tiling_on_sc=False),
        scratch_types=[
            pltpu.VMEM((EPW,), jnp.int32),
            pltpu.VMEM((EPW,), jnp.int32),
            pltpu.VMEM((2 * NB, CH, D), jnp.float32),
            pltpu.VMEM_SHARED((N, D), jnp.float32),
            pltpu.SemaphoreType.DMA,
            pltpu.SemaphoreType.DMA,
            pltpu.SemaphoreType.DMA,
        ],
    )
    def k(hwp_hbm, src_hbm, dst_hbm, out_hbm, sidx_v, didx_v, rows_v, acc_sh,
          gsem, ssem0, ssem1):
        c = lax.axis_index("c")
        s = lax.axis_index("s")
        wid = s * NC + c
        zeros = jnp.zeros((16,), jnp.float32)

        def zero_body(i, carry):
            for j in range(D // 16):
                rows_v[0, i, pl.ds(j * 16, 16)] = zeros
            return carry

        lax.fori_loop(0, CH, zero_body, 0)
        for t in range(5):
            cid = s + NS * t
            base = pl.multiple_of(cid * CH, CH)

            @pl.when(cid < N // CH)
            def _():
                pltpu.sync_copy(rows_v.at[0], acc_sh.at[pl.ds(base, CH)])

            @pl.when(cid == N // CH)
            def _():
                pltpu.sync_copy(rows_v.at[0].at[pl.ds(0, N % CH)],
                                acc_sh.at[pl.ds((N // CH) * CH, N % CH)])

        ebase = wid * EPW
        pltpu.sync_copy(src_hbm.at[pl.ds(ebase, EPW)], sidx_v)
        pltpu.sync_copy(dst_hbm.at[pl.ds(ebase, EPW)], didx_v)
        plsc.subcore_barrier()

        dummy = hwp_hbm.at[pl.ds(0, CH)]

        def fire_gathers(g, s0):
            gb = g * (NB * CH)
            for b in range(NB):
                pltpu.async_copy(
                    hwp_hbm.at[sidx_v.at[pl.ds(gb + b * CH, CH)]],
                    rows_v.at[s0 + b], gsem)

        def drain(sem_, s0):
            for b in range(NB):
                pltpu.make_async_copy(dummy, rows_v.at[s0 + b], sem_).wait()

        def fire_adds(g, s0, sem_):
            gb = g * (NB * CH)
            for b in range(NB):
                pltpu.async_copy(
                    rows_v.at[s0 + b],
                    acc_sh.at[didx_v.at[pl.ds(gb + b * CH, CH)]],
                    sem_, add=True)

        fire_gathers(0, 0)

        def p_body(kk, carry):
            g0 = 2 * kk
            drain(gsem, 0)
            fire_adds(g0, 0, ssem0)

            @pl.when(kk > 0)
            def _():
                drain(ssem1, NB)

            fire_gathers(g0 + 1, NB)
            drain(gsem, NB)
            fire_adds(g0 + 1, NB, ssem1)
            drain(ssem0, 0)

            @pl.when(kk < NPAIR - 1)
            def _():
                fire_gathers(g0 + 2, 0)

            return carry

        lax.fori_loop(0, NPAIR, p_body, 0)
        drain(ssem1, NB)

        tb = NFULL * CH
        pltpu.async_copy(hwp_hbm.at[sidx_v.at[pl.ds(tb, TAIL)]],
                         rows_v.at[0].at[pl.ds(0, TAIL)], gsem).wait()
        pltpu.sync_copy(rows_v.at[0].at[pl.ds(0, TAIL)],
                        acc_sh.at[didx_v.at[pl.ds(tb, TAIL)]], add=True)

        plsc.subcore_barrier()

        for t in range(5):
            cid = s + NS * t
            base = pl.multiple_of(cid * CH, CH)

            @pl.when(cid < N // CH)
            def _():
                pltpu.sync_copy(acc_sh.at[pl.ds(base, CH)],
                                out_hbm.at[pl.ds(c * N + base, CH)])

            @pl.when(cid == N // CH)
            def _():
                tail = N % CH
                t0 = (N // CH) * CH
                pltpu.sync_copy(acc_sh.at[pl.ds(t0, tail)],
                                out_hbm.at[pl.ds(c * N + t0, tail)])

    return k(hwp, src, dst)


def _lrelu(v):
    return jnp.where(v >= 0, v, 0.01 * v)


def _dinv_of(pdeg_blk):
    deg = jnp.sum(pdeg_blk, axis=0) + 1.0
    return lax.rsqrt(deg)[:, None]


def _dot(a, b):
    return jnp.dot(a, b, preferred_element_type=jnp.float32)


def _tc_encoder(x, W1, b1, W2, b2, g1W, pdeg):
    def body(x_r, w1_r, b1_r, w2_r, b2_r, g1w_r, pdeg_r, out_r):
        h = _lrelu(_dot(x_r[...], w1_r[...]) + b1_r[...])
        h = _lrelu(_dot(h, w2_r[...]) + b2_r[...])
        out_r[...] = _dot(h, g1w_r[...]) * _dinv_of(pdeg_r[...])

    full = lambda shape: pl.BlockSpec(shape, lambda i: (0,) * len(shape))
    return pl.pallas_call(
        body,
        grid=(pl.cdiv(N, BN),),
        in_specs=[
            pl.BlockSpec((BN, 128), lambda i: (i, 0)),
            full((128, 128)),
            full((1, 128)),
            full((128, D)),
            full((1, D)),
            full((D, D)),
            pl.BlockSpec((NW, BN), lambda i: (0, i)),
        ],
        out_specs=pl.BlockSpec((BN, D), lambda i: (i, 0)),
        out_shape=jax.ShapeDtypeStruct((N, D), jnp.float32),
    )(x, W1, b1, W2, b2, g1W, pdeg)


def _tc_combine(p0, p1, hwp, pdeg, gW, gb):
    def body(p0_r, p1_r, hwp_r, pdeg_r, gw_r, gb_r, out_r):
        dinv = _dinv_of(pdeg_r[...])
        h = _lrelu((p0_r[...] + p1_r[...] + hwp_r[...]) * dinv + gb_r[...])
        out_r[...] = _dot(h, gw_r[...]) * dinv

    full = lambda shape: pl.BlockSpec(shape, lambda i: (0,) * len(shape))
    blk = pl.BlockSpec((BN, D), lambda i: (i, 0))
    return pl.pallas_call(
        body,
        grid=(pl.cdiv(N, BN),),
        in_specs=[blk, blk, blk,
                  pl.BlockSpec((NW, BN), lambda i: (0, i)),
                  full((D, D)), full((1, D))],
        out_specs=pl.BlockSpec((BN, D), lambda i: (i, 0)),
        out_shape=jax.ShapeDtypeStruct((N, D), jnp.float32),
    )(p0, p1, hwp, pdeg, gW, gb)


def _tc_final(p0, p1, hwp, pdeg, gb, cW, cb):
    def body(p0_r, p1_r, hwp_r, pdeg_r, gb_r, cw_r, cb_r, h_r, log_r):
        dinv = _dinv_of(pdeg_r[...])
        h = _lrelu((p0_r[...] + p1_r[...] + hwp_r[...]) * dinv + gb_r[...])
        h_r[...] = h
        log_r[...] = _dot(h, cw_r[...]) + cb_r[...]

    full = lambda shape: pl.BlockSpec(shape, lambda i: (0,) * len(shape))
    blk = pl.BlockSpec((BN, D), lambda i: (i, 0))
    return pl.pallas_call(
        body,
        grid=(pl.cdiv(N, BN),),
        in_specs=[blk, blk, blk,
                  pl.BlockSpec((NW, BN), lambda i: (0, i)),
                  full((1, D)), full((D, 2)), full((1, 2))],
        out_specs=[pl.BlockSpec((BN, D), lambda i: (i, 0)),
                   pl.BlockSpec((BN, 2), lambda i: (i, 0))],
        out_shape=[jax.ShapeDtypeStruct((N, D), jnp.float32),
                   jax.ShapeDtypeStruct((N, 2), jnp.float32)],
    )(p0, p1, hwp, pdeg, gb, cW, cb)


def kernel(x, edge_index, W1, b1, W2, b2, g1W, g1b, g2W, g2b, cW, cb):
    src = edge_index[0]
    dst = edge_index[1]
    pdeg = _sc_deg(dst)
    hw1p = _tc_encoder(x, W1, b1.reshape(1, -1), W2, b2.reshape(1, -1), g1W, pdeg)
    part1 = _sc_scatter(hw1p, src, dst)
    hw2p = _tc_combine(part1[:N], part1[N:], hw1p, pdeg, g2W, g1b.reshape(1, -1))
    part2 = _sc_scatter(hw2p, src, dst)
    h, logits = _tc_final(part2[:N], part2[N:], hw2p, pdeg, g2b.reshape(1, -1),
                          cW, cb.reshape(1, -1))
    return (logits, h)


# speedup vs baseline: 43.3362x; 1.1041x over previous
import functools
import jax
import jax.numpy as jnp
from jax import lax
from jax.experimental import pallas as pl
from jax.experimental.pallas import tpu as pltpu
from jax.experimental.pallas import tpu_sc as plsc

N = 10000
E = 320000
D = 64
NC = 2
NS = 16
NW = NC * NS
CH = 128
EPW = E // NW
NFULL = EPW // CH
TAIL = EPW - NFULL * CH
NB = 3
NPAIR = NFULL // (2 * NB)

BNE = 1024
BN = 1000
NBLK = N // BN

_mesh = lambda: plsc.VectorSubcoreMesh(core_axis_name="c", subcore_axis_name="s")


def _sc_deg(edge_index):
    @functools.partial(
        pl.kernel,
        out_type=jax.ShapeDtypeStruct((NW, N), jnp.float32),
        mesh=_mesh(),
        compiler_params=pltpu.CompilerParams(needs_layout_passes=False, use_tc_tiling_on_sc=False),
        scratch_types=[
            pltpu.VMEM((N,), jnp.float32),
            pltpu.VMEM((EPW,), jnp.int32),
        ],
    )
    def k(ei_hbm, pdeg_hbm, deg_v, idx_v):
        c = lax.axis_index("c")
        s = lax.axis_index("s")
        wid = s * NC + c
        zeros = jnp.zeros((16,), jnp.float32)
        ones = jnp.ones((16,), jnp.float32)

        def zero_body(i, carry):
            deg_v[pl.ds(i * 16, 16)] = zeros
            return carry

        lax.fori_loop(0, N // 16, zero_body, 0)

        pltpu.sync_copy(ei_hbm.at[1].at[pl.ds(wid * EPW, EPW)], idx_v)

        def h_body(i, carry):
            for j in range(25):
                idx = idx_v[pl.ds((i * 25 + j) * 16, 16)]
                plsc.addupdate_scatter(deg_v, [idx], ones)
            return carry

        lax.fori_loop(0, EPW // (25 * 16), h_body, 0)

        pltpu.sync_copy(deg_v, pdeg_hbm.at[wid])

    return k(edge_index)


def _sc_scatter(hwp, edge_index):
    @functools.partial(
        pl.kernel,
        out_type=jax.ShapeDtypeStruct((NC * N, D), jnp.float32),
        mesh=_mesh(),
        compiler_params=pltpu.CompilerParams(needs_layout_passes=False, use_tc_tiling_on_sc=False),
        scratch_types=[
            pltpu.VMEM((EPW,), jnp.int32),
            pltpu.VMEM((EPW,), jnp.int32),
            pltpu.VMEM((2 * NB, CH, D), jnp.float32),
            pltpu.VMEM_SHARED((N, D), jnp.float32),
            pltpu.SemaphoreType.DMA,
            pltpu.SemaphoreType.DMA,
            pltpu.SemaphoreType.DMA,
        ],
    )
    def k(hwp_hbm, ei_hbm, out_hbm, sidx_v, didx_v, rows_v, acc_sh,
          gsem, ssem0, ssem1):
        c = lax.axis_index("c")
        s = lax.axis_index("s")
        wid = s * NC + c
        zeros = jnp.zeros((16,), jnp.float32)

        def zero_body(i, carry):
            for j in range(D // 16):
                rows_v[0, i, pl.ds(j * 16, 16)] = zeros
            return carry

        lax.fori_loop(0, CH, zero_body, 0)
        for t in range(5):
            cid = s + NS * t
            base = pl.multiple_of(cid * CH, CH)

            @pl.when(cid < N // CH)
            def _():
                pltpu.sync_copy(rows_v.at[0], acc_sh.at[pl.ds(base, CH)])

            @pl.when(cid == N // CH)
            def _():
                pltpu.sync_copy(rows_v.at[0].at[pl.ds(0, N % CH)],
                                acc_sh.at[pl.ds((N // CH) * CH, N % CH)])

        ebase = wid * EPW
        pltpu.sync_copy(ei_hbm.at[0].at[pl.ds(ebase, EPW)], sidx_v)
        pltpu.sync_copy(ei_hbm.at[1].at[pl.ds(ebase, EPW)], didx_v)
        plsc.subcore_barrier()

        dummy = hwp_hbm.at[pl.ds(0, CH)]

        def fire_gathers(g, s0):
            gb = g * (NB * CH)
            for b in range(NB):
                pltpu.async_copy(
                    hwp_hbm.at[sidx_v.at[pl.ds(gb + b * CH, CH)]],
                    rows_v.at[s0 + b], gsem)

        def drain(sem_, s0):
            for b in range(NB):
                pltpu.make_async_copy(dummy, rows_v.at[s0 + b], sem_).wait()

        def fire_adds(g, s0, sem_):
            gb = g * (NB * CH)
            for b in range(NB):
                pltpu.async_copy(
                    rows_v.at[s0 + b],
                    acc_sh.at[didx_v.at[pl.ds(gb + b * CH, CH)]],
                    sem_, add=True)

        fire_gathers(0, 0)

        def p_body(kk, carry):
            g0 = 2 * kk
            drain(gsem, 0)
            fire_adds(g0, 0, ssem0)

            @pl.when(kk > 0)
            def _():
                drain(ssem1, NB)

            fire_gathers(g0 + 1, NB)
            drain(gsem, NB)
            fire_adds(g0 + 1, NB, ssem1)
            drain(ssem0, 0)

            @pl.when(kk < NPAIR - 1)
            def _():
                fire_gathers(g0 + 2, 0)

            return carry

        lax.fori_loop(0, NPAIR, p_body, 0)
        drain(ssem1, NB)

        tb = NFULL * CH
        pltpu.async_copy(hwp_hbm.at[sidx_v.at[pl.ds(tb, TAIL)]],
                         rows_v.at[0].at[pl.ds(0, TAIL)], gsem).wait()
        pltpu.sync_copy(rows_v.at[0].at[pl.ds(0, TAIL)],
                        acc_sh.at[didx_v.at[pl.ds(tb, TAIL)]], add=True)

        plsc.subcore_barrier()

        for t in range(5):
            cid = s + NS * t
            base = pl.multiple_of(cid * CH, CH)

            @pl.when(cid < N // CH)
            def _():
                pltpu.sync_copy(acc_sh.at[pl.ds(base, CH)],
                                out_hbm.at[pl.ds(c * N + base, CH)])

            @pl.when(cid == N // CH)
            def _():
                tail = N % CH
                t0 = (N // CH) * CH
                pltpu.sync_copy(acc_sh.at[pl.ds(t0, tail)],
                                out_hbm.at[pl.ds(c * N + t0, tail)])

    return k(hwp, edge_index)


def _lrelu(v):
    return jnp.where(v >= 0, v, 0.01 * v)


def _dinv_of(pdeg_blk):
    deg = jnp.sum(pdeg_blk, axis=0) + 1.0
    return lax.rsqrt(deg)[:, None]


def _dot(a, b):
    return jnp.dot(a, b, preferred_element_type=jnp.float32)


def _tc_encoder(x, W1, b1, W2, b2, g1W, pdeg):
    def body(x_r, w1_r, b1_r, w2_r, b2_r, g1w_r, pdeg_r, out_r, dinv_r):
        h = _lrelu(_dot(x_r[...], w1_r[...]) + b1_r[...])
        h = _lrelu(_dot(h, w2_r[...]) + b2_r[...])
        dinv = _dinv_of(pdeg_r[...])
        out_r[...] = _dot(h, g1w_r[...]) * dinv
        dinv_r[...] = dinv

    full = lambda shape: pl.BlockSpec(shape, lambda i: (0,) * len(shape))
    return pl.pallas_call(
        body,
        grid=(pl.cdiv(N, BNE),),
        in_specs=[
            pl.BlockSpec((BNE, 128), lambda i: (i, 0)),
            full((128, 128)),
            full((1, 128)),
            full((128, D)),
            full((1, D)),
            full((D, D)),
            pl.BlockSpec((NW, BNE), lambda i: (0, i)),
        ],
        out_specs=[pl.BlockSpec((BNE, D), lambda i: (i, 0)),
                   pl.BlockSpec((BNE, 1), lambda i: (i, 0))],
        out_shape=[jax.ShapeDtypeStruct((N, D), jnp.float32),
                   jax.ShapeDtypeStruct((N, 1), jnp.float32)],
    )(x, W1, b1, W2, b2, g1W, pdeg)


def _tc_combine(part, hwp, dinv, gW, gb):
    def body(p0_r, p1_r, hwp_r, dinv_r, gw_r, gb_r, out_r):
        dinv = dinv_r[...]
        h = _lrelu((p0_r[...] + p1_r[...] + hwp_r[...]) * dinv + gb_r[...])
        out_r[...] = _dot(h, gw_r[...]) * dinv

    full = lambda shape: pl.BlockSpec(shape, lambda i: (0,) * len(shape))
    blk = pl.BlockSpec((BN, D), lambda i: (i, 0))
    return pl.pallas_call(
        body,
        grid=(NBLK,),
        in_specs=[pl.BlockSpec((BN, D), lambda i: (i, 0)),
                  pl.BlockSpec((BN, D), lambda i: (i + NBLK, 0)),
                  blk,
                  pl.BlockSpec((BN, 1), lambda i: (i, 0)),
                  full((D, D)), full((1, D))],
        out_specs=pl.BlockSpec((BN, D), lambda i: (i, 0)),
        out_shape=jax.ShapeDtypeStruct((N, D), jnp.float32),
    )(part, part, hwp, dinv, gW, gb)


def _tc_final(part, hwp, dinv, gb, cW, cb):
    def body(p0_r, p1_r, hwp_r, dinv_r, gb_r, cw_r, cb_r, h_r, log_r):
        dinv = dinv_r[...]
        h = _lrelu((p0_r[...] + p1_r[...] + hwp_r[...]) * dinv + gb_r[...])
        h_r[...] = h
        log_r[...] = _dot(h, cw_r[...]) + cb_r[...]

    full = lambda shape: pl.BlockSpec(shape, lambda i: (0,) * len(shape))
    return pl.pallas_call(
        body,
        grid=(NBLK,),
        in_specs=[pl.BlockSpec((BN, D), lambda i: (i, 0)),
                  pl.BlockSpec((BN, D), lambda i: (i + NBLK, 0)),
                  pl.BlockSpec((BN, D), lambda i: (i, 0)),
                  pl.BlockSpec((BN, 1), lambda i: (i, 0)),
                  full((1, D)), full((D, 2)), full((1, 2))],
        out_specs=[pl.BlockSpec((BN, D), lambda i: (i, 0)),
                   pl.BlockSpec((BN, 2), lambda i: (i, 0))],
        out_shape=[jax.ShapeDtypeStruct((N, D), jnp.float32),
                   jax.ShapeDtypeStruct((N, 2), jnp.float32)],
    )(part, part, hwp, dinv, gb, cW, cb)


def kernel(x, edge_index, W1, b1, W2, b2, g1W, g1b, g2W, g2b, cW, cb):
    pdeg = _sc_deg(edge_index)
    hw1p, dinv = _tc_encoder(x, W1, b1.reshape(1, -1), W2, b2.reshape(1, -1),
                             g1W, pdeg)
    part1 = _sc_scatter(hw1p, edge_index)
    hw2p = _tc_combine(part1, hw1p, dinv, g2W, g1b.reshape(1, -1))
    part2 = _sc_scatter(hw2p, edge_index)
    h, logits = _tc_final(part2, hw2p, dinv, g2b.reshape(1, -1),
                          cW, cb.reshape(1, -1))
    return (logits, h)


# speedup vs baseline: 44.5758x; 1.0286x over previous
import functools
import jax
import jax.numpy as jnp
from jax import lax
from jax.experimental import pallas as pl
from jax.experimental.pallas import tpu as pltpu
from jax.experimental.pallas import tpu_sc as plsc

N = 10000
E = 320000
D = 64
NC = 2
NS = 16
NW = NC * NS
CH = 128
EPW = E // NW
NFULL = EPW // CH
TAIL = EPW - NFULL * CH
NB = 3
NPAIR = NFULL // (2 * NB)

BNE = 2048
BN = 2000
NBLK = N // BN

_mesh = lambda: plsc.VectorSubcoreMesh(core_axis_name="c", subcore_axis_name="s")


def _sc_deg(edge_index):
    @functools.partial(
        pl.kernel,
        out_type=jax.ShapeDtypeStruct((NW, N), jnp.float32),
        mesh=_mesh(),
        compiler_params=pltpu.CompilerParams(needs_layout_passes=False, use_tc_tiling_on_sc=False),
        scratch_types=[
            pltpu.VMEM((N,), jnp.float32),
            pltpu.VMEM((EPW,), jnp.int32),
        ],
    )
    def k(ei_hbm, pdeg_hbm, deg_v, idx_v):
        c = lax.axis_index("c")
        s = lax.axis_index("s")
        wid = s * NC + c
        zeros = jnp.zeros((16,), jnp.float32)
        ones = jnp.ones((16,), jnp.float32)

        def zero_body(i, carry):
            deg_v[pl.ds(i * 16, 16)] = zeros
            return carry

        lax.fori_loop(0, N // 16, zero_body, 0)

        pltpu.sync_copy(ei_hbm.at[1].at[pl.ds(wid * EPW, EPW)], idx_v)

        def h_body(i, carry):
            for j in range(25):
                idx = idx_v[pl.ds((i * 25 + j) * 16, 16)]
                plsc.addupdate_scatter(deg_v, [idx], ones)
            return carry

        lax.fori_loop(0, EPW // (25 * 16), h_body, 0)

        pltpu.sync_copy(deg_v, pdeg_hbm.at[wid])

    return k(edge_index)


def _sc_scatter(hwp, edge_index):
    @functools.partial(
        pl.kernel,
        out_type=jax.ShapeDtypeStruct((NC * N, D), jnp.float32),
        mesh=_mesh(),
        compiler_params=pltpu.CompilerParams(needs_layout_passes=False, use_tc_tiling_on_sc=False),
        scratch_types=[
            pltpu.VMEM((EPW,), jnp.int32),
            pltpu.VMEM((EPW,), jnp.int32),
            pltpu.VMEM((2 * NB, CH, D), jnp.float32),
            pltpu.VMEM_SHARED((N, D), jnp.float32),
            pltpu.SemaphoreType.DMA,
            pltpu.SemaphoreType.DMA,
            pltpu.SemaphoreType.DMA,
        ],
    )
    def k(hwp_hbm, ei_hbm, out_hbm, sidx_v, didx_v, rows_v, acc_sh,
          gsem, ssem0, ssem1):
        c = lax.axis_index("c")
        s = lax.axis_index("s")
        wid = s * NC + c
        zeros = jnp.zeros((16,), jnp.float32)

        def zero_body(i, carry):
            for j in range(D // 16):
                rows_v[0, i, pl.ds(j * 16, 16)] = zeros
            return carry

        lax.fori_loop(0, CH, zero_body, 0)
        for t in range(5):
            cid = s + NS * t
            base = pl.multiple_of(cid * CH, CH)

            @pl.when(cid < N // CH)
            def _():
                pltpu.sync_copy(rows_v.at[0], acc_sh.at[pl.ds(base, CH)])

            @pl.when(cid == N // CH)
            def _():
                pltpu.sync_copy(rows_v.at[0].at[pl.ds(0, N % CH)],
                                acc_sh.at[pl.ds((N // CH) * CH, N % CH)])

        ebase = wid * EPW
        pltpu.sync_copy(ei_hbm.at[0].at[pl.ds(ebase, EPW)], sidx_v)
        pltpu.sync_copy(ei_hbm.at[1].at[pl.ds(ebase, EPW)], didx_v)
        plsc.subcore_barrier()

        dummy = hwp_hbm.at[pl.ds(0, CH)]

        def fire_gathers(g, s0):
            gb = g * (NB * CH)
            for b in range(NB):
                pltpu.async_copy(
                    hwp_hbm.at[sidx_v.at[pl.ds(gb + b * CH, CH)]],
                    rows_v.at[s0 + b], gsem)

        def drain(sem_, s0):
            for b in range(NB):
                pltpu.make_async_copy(dummy, rows_v.at[s0 + b], sem_).wait()

        def fire_adds(g, s0, sem_):
            gb = g * (NB * CH)
            for b in range(NB):
                pltpu.async_copy(
                    rows_v.at[s0 + b],
                    acc_sh.at[didx_v.at[pl.ds(gb + b * CH, CH)]],
                    sem_, add=True)

        fire_gathers(0, 0)

        def p_body(kk, carry):
            g0 = 2 * kk
            drain(gsem, 0)
            fire_adds(g0, 0, ssem0)

            @pl.when(kk > 0)
            def _():
                drain(ssem1, NB)

            fire_gathers(g0 + 1, NB)
            drain(gsem, NB)
            fire_adds(g0 + 1, NB, ssem1)
            drain(ssem0, 0)

            @pl.when(kk < NPAIR - 1)
            def _():
                fire_gathers(g0 + 2, 0)

            return carry

        lax.fori_loop(0, NPAIR, p_body, 0)
        drain(ssem1, NB)

        tb = NFULL * CH
        pltpu.async_copy(hwp_hbm.at[sidx_v.at[pl.ds(tb, TAIL)]],
                         rows_v.at[0].at[pl.ds(0, TAIL)], gsem).wait()
        pltpu.sync_copy(rows_v.at[0].at[pl.ds(0, TAIL)],
                        acc_sh.at[didx_v.at[pl.ds(tb, TAIL)]], add=True)

        plsc.subcore_barrier()

        for t in range(5):
            cid = s + NS * t
            base = pl.multiple_of(cid * CH, CH)

            @pl.when(cid < N // CH)
            def _():
                pltpu.sync_copy(acc_sh.at[pl.ds(base, CH)],
                                out_hbm.at[pl.ds(c * N + base, CH)])

            @pl.when(cid == N // CH)
            def _():
                tail = N % CH
                t0 = (N // CH) * CH
                pltpu.sync_copy(acc_sh.at[pl.ds(t0, tail)],
                                out_hbm.at[pl.ds(c * N + t0, tail)])

    return k(hwp, edge_index)


def _lrelu(v):
    return jnp.where(v >= 0, v, 0.01 * v)


def _dinv_of(pdeg_blk):
    deg = jnp.sum(pdeg_blk, axis=0) + 1.0
    return lax.rsqrt(deg)[:, None]


def _dot(a, b):
    return jnp.dot(a, b, preferred_element_type=jnp.float32)


def _tc_encoder(x, W1, b1, W2, b2, g1W, pdeg):
    def body(x_r, w1_r, b1_r, w2_r, b2_r, g1w_r, pdeg_r, out_r, dinv_r):
        h = _lrelu(_dot(x_r[...], w1_r[...]) + b1_r[...])
        h = _lrelu(_dot(h, w2_r[...]) + b2_r[...])
        dinv = _dinv_of(pdeg_r[...])
        out_r[...] = _dot(h, g1w_r[...]) * dinv
        dinv_r[...] = dinv

    full = lambda shape: pl.BlockSpec(shape, lambda i: (0,) * len(shape))
    return pl.pallas_call(
        body,
        grid=(pl.cdiv(N, BNE),),
        in_specs=[
            pl.BlockSpec((BNE, 128), lambda i: (i, 0)),
            full((128, 128)),
            full((1, 128)),
            full((128, D)),
            full((1, D)),
            full((D, D)),
            pl.BlockSpec((NW, BNE), lambda i: (0, i)),
        ],
        out_specs=[pl.BlockSpec((BNE, D), lambda i: (i, 0)),
                   pl.BlockSpec((BNE, 1), lambda i: (i, 0))],
        out_shape=[jax.ShapeDtypeStruct((N, D), jnp.float32),
                   jax.ShapeDtypeStruct((N, 1), jnp.float32)],
    )(x, W1, b1, W2, b2, g1W, pdeg)


def _tc_combine(part, hwp, dinv, gW, gb):
    def body(p0_r, p1_r, hwp_r, dinv_r, gw_r, gb_r, out_r):
        dinv = dinv_r[...]
        h = _lrelu((p0_r[...] + p1_r[...] + hwp_r[...]) * dinv + gb_r[...])
        out_r[...] = _dot(h, gw_r[...]) * dinv

    full = lambda shape: pl.BlockSpec(shape, lambda i: (0,) * len(shape))
    blk = pl.BlockSpec((BN, D), lambda i: (i, 0))
    return pl.pallas_call(
        body,
        grid=(NBLK,),
        in_specs=[pl.BlockSpec((BN, D), lambda i: (i, 0)),
                  pl.BlockSpec((BN, D), lambda i: (i + NBLK, 0)),
                  blk,
                  pl.BlockSpec((BN, 1), lambda i: (i, 0)),
                  full((D, D)), full((1, D))],
        out_specs=pl.BlockSpec((BN, D), lambda i: (i, 0)),
        out_shape=jax.ShapeDtypeStruct((N, D), jnp.float32),
    )(part, part, hwp, dinv, gW, gb)


def _tc_final(part, hwp, dinv, gb, cW, cb):
    def body(p0_r, p1_r, hwp_r, dinv_r, gb_r, cw_r, cb_r, h_r, log_r):
        dinv = dinv_r[...]
        h = _lrelu((p0_r[...] + p1_r[...] + hwp_r[...]) * dinv + gb_r[...])
        h_r[...] = h
        log_r[...] = _dot(h, cw_r[...]) + cb_r[...]

    full = lambda shape: pl.BlockSpec(shape, lambda i: (0,) * len(shape))
    return pl.pallas_call(
        body,
        grid=(NBLK,),
        in_specs=[pl.BlockSpec((BN, D), lambda i: (i, 0)),
                  pl.BlockSpec((BN, D), lambda i: (i + NBLK, 0)),
                  pl.BlockSpec((BN, D), lambda i: (i, 0)),
                  pl.BlockSpec((BN, 1), lambda i: (i, 0)),
                  full((1, D)), full((D, 2)), full((1, 2))],
        out_specs=[pl.BlockSpec((BN, D), lambda i: (i, 0)),
                   pl.BlockSpec((BN, 2), lambda i: (i, 0))],
        out_shape=[jax.ShapeDtypeStruct((N, D), jnp.float32),
                   jax.ShapeDtypeStruct((N, 2), jnp.float32)],
    )(part, part, hwp, dinv, gb, cW, cb)


def kernel(x, edge_index, W1, b1, W2, b2, g1W, g1b, g2W, g2b, cW, cb):
    pdeg = _sc_deg(edge_index)
    hw1p, dinv = _tc_encoder(x, W1, b1.reshape(1, -1), W2, b2.reshape(1, -1),
                             g1W, pdeg)
    part1 = _sc_scatter(hw1p, edge_index)
    hw2p = _tc_combine(part1, hw1p, dinv, g2W, g1b.reshape(1, -1))
    part2 = _sc_scatter(hw2p, edge_index)
    h, logits = _tc_final(part2, hw2p, dinv, g2b.reshape(1, -1),
                          cW, cb.reshape(1, -1))
    return (logits, h)


# speedup vs baseline: 45.2370x; 1.0148x over previous
import functools
import jax
import jax.numpy as jnp
from jax import lax
from jax.experimental import pallas as pl
from jax.experimental.pallas import tpu as pltpu
from jax.experimental.pallas import tpu_sc as plsc

N = 10000
E = 320000
D = 64
NC = 2
NS = 16
NW = NC * NS
CH = 128
EPW = E // NW
NFULL = EPW // CH
TAIL = EPW - NFULL * CH
NB = 4
NPAIR = NFULL // (2 * NB)
NREM = NFULL - 2 * NPAIR * NB

BNE = 2048
BN = 2000
NBLK = N // BN

_mesh = lambda: plsc.VectorSubcoreMesh(core_axis_name="c", subcore_axis_name="s")


def _sc_deg(edge_index):
    @functools.partial(
        pl.kernel,
        out_type=jax.ShapeDtypeStruct((NW, N), jnp.float32),
        mesh=_mesh(),
        compiler_params=pltpu.CompilerParams(needs_layout_passes=False, use_tc_tiling_on_sc=False),
        scratch_types=[
            pltpu.VMEM((N,), jnp.float32),
            pltpu.VMEM((EPW,), jnp.int32),
        ],
    )
    def k(ei_hbm, pdeg_hbm, deg_v, idx_v):
        c = lax.axis_index("c")
        s = lax.axis_index("s")
        wid = s * NC + c
        zeros = jnp.zeros((16,), jnp.float32)
        ones = jnp.ones((16,), jnp.float32)

        def zero_body(i, carry):
            deg_v[pl.ds(i * 16, 16)] = zeros
            return carry

        lax.fori_loop(0, N // 16, zero_body, 0)

        pltpu.sync_copy(ei_hbm.at[1].at[pl.ds(wid * EPW, EPW)], idx_v)

        def h_body(i, carry):
            for j in range(25):
                idx = idx_v[pl.ds((i * 25 + j) * 16, 16)]
                plsc.addupdate_scatter(deg_v, [idx], ones)
            return carry

        lax.fori_loop(0, EPW // (25 * 16), h_body, 0)

        pltpu.sync_copy(deg_v, pdeg_hbm.at[wid])

    return k(edge_index)


def _sc_scatter(hwp, edge_index):
    @functools.partial(
        pl.kernel,
        out_type=jax.ShapeDtypeStruct((NC * N, D), jnp.float32),
        mesh=_mesh(),
        compiler_params=pltpu.CompilerParams(needs_layout_passes=False, use_tc_tiling_on_sc=False),
        scratch_types=[
            pltpu.VMEM((EPW,), jnp.int32),
            pltpu.VMEM((EPW,), jnp.int32),
            pltpu.VMEM((2 * NB, CH, D), jnp.float32),
            pltpu.VMEM_SHARED((N, D), jnp.float32),
            pltpu.SemaphoreType.DMA,
            pltpu.SemaphoreType.DMA,
            pltpu.SemaphoreType.DMA,
        ],
    )
    def k(hwp_hbm, ei_hbm, out_hbm, sidx_v, didx_v, rows_v, acc_sh,
          gsem, ssem0, ssem1):
        c = lax.axis_index("c")
        s = lax.axis_index("s")
        wid = s * NC + c
        zeros = jnp.zeros((16,), jnp.float32)

        def zero_body(i, carry):
            for j in range(D // 16):
                rows_v[0, i, pl.ds(j * 16, 16)] = zeros
            return carry

        lax.fori_loop(0, CH, zero_body, 0)
        for t in range(5):
            cid = s + NS * t
            base = pl.multiple_of(cid * CH, CH)

            @pl.when(cid < N // CH)
            def _():
                pltpu.sync_copy(rows_v.at[0], acc_sh.at[pl.ds(base, CH)])

            @pl.when(cid == N // CH)
            def _():
                pltpu.sync_copy(rows_v.at[0].at[pl.ds(0, N % CH)],
                                acc_sh.at[pl.ds((N // CH) * CH, N % CH)])

        ebase = wid * EPW
        pltpu.sync_copy(ei_hbm.at[0].at[pl.ds(ebase, EPW)], sidx_v)
        pltpu.sync_copy(ei_hbm.at[1].at[pl.ds(ebase, EPW)], didx_v)
        plsc.subcore_barrier()

        dummy = hwp_hbm.at[pl.ds(0, CH)]

        def fire_gathers(g, s0, nb=NB):
            gb = g * (NB * CH)
            for b in range(nb):
                pltpu.async_copy(
                    hwp_hbm.at[sidx_v.at[pl.ds(gb + b * CH, CH)]],
                    rows_v.at[s0 + b], gsem)

        def drain(sem_, s0, nb=NB):
            for b in range(nb):
                pltpu.make_async_copy(dummy, rows_v.at[s0 + b], sem_).wait()

        def fire_adds(g, s0, sem_, nb=NB):
            gb = g * (NB * CH)
            for b in range(nb):
                pltpu.async_copy(
                    rows_v.at[s0 + b],
                    acc_sh.at[didx_v.at[pl.ds(gb + b * CH, CH)]],
                    sem_, add=True)

        fire_gathers(0, 0)

        def p_body(kk, carry):
            g0 = 2 * kk
            drain(gsem, 0)
            fire_adds(g0, 0, ssem0)

            @pl.when(kk > 0)
            def _():
                drain(ssem1, NB)

            fire_gathers(g0 + 1, NB)
            drain(gsem, NB)
            fire_adds(g0 + 1, NB, ssem1)
            drain(ssem0, 0)

            @pl.when(kk < NPAIR - 1)
            def _():
                fire_gathers(g0 + 2, 0)

            return carry

        lax.fori_loop(0, NPAIR, p_body, 0)
        ge = 2 * NPAIR
        fire_gathers(ge, 0)
        drain(gsem, 0)
        fire_adds(ge, 0, ssem0)
        drain(ssem1, NB)
        fire_gathers(ge + 1, NB, NREM - NB)
        drain(gsem, NB, NREM - NB)
        fire_adds(ge + 1, NB, ssem1, NREM - NB)
        drain(ssem0, 0)
        drain(ssem1, NB, NREM - NB)

        tb = NFULL * CH
        pltpu.async_copy(hwp_hbm.at[sidx_v.at[pl.ds(tb, TAIL)]],
                         rows_v.at[0].at[pl.ds(0, TAIL)], gsem).wait()
        pltpu.sync_copy(rows_v.at[0].at[pl.ds(0, TAIL)],
                        acc_sh.at[didx_v.at[pl.ds(tb, TAIL)]], add=True)

        plsc.subcore_barrier()

        for t in range(5):
            cid = s + NS * t
            base = pl.multiple_of(cid * CH, CH)

            @pl.when(cid < N // CH)
            def _():
                pltpu.sync_copy(acc_sh.at[pl.ds(base, CH)],
                                out_hbm.at[pl.ds(c * N + base, CH)])

            @pl.when(cid == N // CH)
            def _():
                tail = N % CH
                t0 = (N // CH) * CH
                pltpu.sync_copy(acc_sh.at[pl.ds(t0, tail)],
                                out_hbm.at[pl.ds(c * N + t0, tail)])

    return k(hwp, edge_index)


def _lrelu(v):
    return jnp.where(v >= 0, v, 0.01 * v)


def _dinv_of(pdeg_blk):
    deg = jnp.sum(pdeg_blk, axis=0) + 1.0
    return lax.rsqrt(deg)[:, None]


def _dot(a, b):
    return jnp.dot(a, b, preferred_element_type=jnp.float32)


def _tc_encoder(x, W1, b1, W2, b2, g1W, pdeg):
    def body(x_r, w1_r, b1_r, w2_r, b2_r, g1w_r, pdeg_r, out_r, dinv_r):
        h = _lrelu(_dot(x_r[...], w1_r[...]) + b1_r[...])
        h = _lrelu(_dot(h, w2_r[...]) + b2_r[...])
        dinv = _dinv_of(pdeg_r[...])
        out_r[...] = _dot(h, g1w_r[...]) * dinv
        dinv_r[...] = dinv

    full = lambda shape: pl.BlockSpec(shape, lambda i: (0,) * len(shape))
    return pl.pallas_call(
        body,
        grid=(pl.cdiv(N, BNE),),
        in_specs=[
            pl.BlockSpec((BNE, 128), lambda i: (i, 0)),
            full((128, 128)),
            full((1, 128)),
            full((128, D)),
            full((1, D)),
            full((D, D)),
            pl.BlockSpec((NW, BNE), lambda i: (0, i)),
        ],
        out_specs=[pl.BlockSpec((BNE, D), lambda i: (i, 0)),
                   pl.BlockSpec((BNE, 1), lambda i: (i, 0))],
        out_shape=[jax.ShapeDtypeStruct((N, D), jnp.float32),
                   jax.ShapeDtypeStruct((N, 1), jnp.float32)],
    )(x, W1, b1, W2, b2, g1W, pdeg)


def _tc_combine(part, hwp, dinv, gW, gb):
    def body(p0_r, p1_r, hwp_r, dinv_r, gw_r, gb_r, out_r):
        dinv = dinv_r[...]
        h = _lrelu((p0_r[...] + p1_r[...] + hwp_r[...]) * dinv + gb_r[...])
        out_r[...] = _dot(h, gw_r[...]) * dinv

    full = lambda shape: pl.BlockSpec(shape, lambda i: (0,) * len(shape))
    blk = pl.BlockSpec((BN, D), lambda i: (i, 0))
    return pl.pallas_call(
        body,
        grid=(NBLK,),
        in_specs=[pl.BlockSpec((BN, D), lambda i: (i, 0)),
                  pl.BlockSpec((BN, D), lambda i: (i + NBLK, 0)),
                  blk,
                  pl.BlockSpec((BN, 1), lambda i: (i, 0)),
                  full((D, D)), full((1, D))],
        out_specs=pl.BlockSpec((BN, D), lambda i: (i, 0)),
        out_shape=jax.ShapeDtypeStruct((N, D), jnp.float32),
    )(part, part, hwp, dinv, gW, gb)


def _tc_final(part, hwp, dinv, gb, cW, cb):
    def body(p0_r, p1_r, hwp_r, dinv_r, gb_r, cw_r, cb_r, h_r, log_r):
        dinv = dinv_r[...]
        h = _lrelu((p0_r[...] + p1_r[...] + hwp_r[...]) * dinv + gb_r[...])
        h_r[...] = h
        log_r[...] = _dot(h, cw_r[...]) + cb_r[...]

    full = lambda shape: pl.BlockSpec(shape, lambda i: (0,) * len(shape))
    return pl.pallas_call(
        body,
        grid=(NBLK,),
        in_specs=[pl.BlockSpec((BN, D), lambda i: (i, 0)),
                  pl.BlockSpec((BN, D), lambda i: (i + NBLK, 0)),
                  pl.BlockSpec((BN, D), lambda i: (i, 0)),
                  pl.BlockSpec((BN, 1), lambda i: (i, 0)),
                  full((1, D)), full((D, 2)), full((1, 2))],
        out_specs=[pl.BlockSpec((BN, D), lambda i: (i, 0)),
                   pl.BlockSpec((BN, 2), lambda i: (i, 0))],
        out_shape=[jax.ShapeDtypeStruct((N, D), jnp.float32),
                   jax.ShapeDtypeStruct((N, 2), jnp.float32)],
    )(part, part, hwp, dinv, gb, cW, cb)


def kernel(x, edge_index, W1, b1, W2, b2, g1W, g1b, g2W, g2b, cW, cb):
    pdeg = _sc_deg(edge_index)
    hw1p, dinv = _tc_encoder(x, W1, b1.reshape(1, -1), W2, b2.reshape(1, -1),
                             g1W, pdeg)
    part1 = _sc_scatter(hw1p, edge_index)
    hw2p = _tc_combine(part1, hw1p, dinv, g2W, g1b.reshape(1, -1))
    part2 = _sc_scatter(hw2p, edge_index)
    h, logits = _tc_final(part2, hw2p, dinv, g2b.reshape(1, -1),
                          cW, cb.reshape(1, -1))
    return (logits, h)


# speedup vs baseline: 46.1572x; 1.0203x over previous
import functools
import jax
import jax.numpy as jnp
from jax import lax
from jax.experimental import pallas as pl
from jax.experimental.pallas import tpu as pltpu
from jax.experimental.pallas import tpu_sc as plsc

N = 10000
E = 320000
D = 64
NC = 2
NS = 16
NW = NC * NS
CH = 128
EPW = E // NW
NFULL = EPW // CH
TAIL = EPW - NFULL * CH
NB = 4
NPAIR = NFULL // (2 * NB)
NREM = NFULL - 2 * NPAIR * NB

BNE = 2048
BN = 2000
NBLK = N // BN

_mesh = lambda: plsc.VectorSubcoreMesh(core_axis_name="c", subcore_axis_name="s")


def _sc_deg(edge_index):
    @functools.partial(
        pl.kernel,
        out_type=jax.ShapeDtypeStruct((NW, N), jnp.float32),
        mesh=_mesh(),
        compiler_params=pltpu.CompilerParams(needs_layout_passes=False, use_tc_tiling_on_sc=False),
        scratch_types=[
            pltpu.VMEM((N,), jnp.float32),
            pltpu.VMEM((EPW,), jnp.int32),
        ],
    )
    def k(ei_hbm, pdeg_hbm, deg_v, idx_v):
        c = lax.axis_index("c")
        s = lax.axis_index("s")
        wid = s * NC + c
        zeros = jnp.zeros((16,), jnp.float32)
        ones = jnp.ones((16,), jnp.float32)

        def zero_body(i, carry):
            deg_v[pl.ds(i * 16, 16)] = zeros
            return carry

        lax.fori_loop(0, N // 16, zero_body, 0)

        pltpu.sync_copy(ei_hbm.at[1].at[pl.ds(wid * EPW, EPW)], idx_v)

        def h_body(i, carry):
            for j in range(25):
                idx = idx_v[pl.ds((i * 25 + j) * 16, 16)]
                plsc.addupdate_scatter(deg_v, [idx], ones)
            return carry

        lax.fori_loop(0, EPW // (25 * 16), h_body, 0)

        pltpu.sync_copy(deg_v, pdeg_hbm.at[wid])

    return k(edge_index)


def _sc_scatter(hwp, edge_index):
    @functools.partial(
        pl.kernel,
        out_type=jax.ShapeDtypeStruct((NC * N, D), jnp.float32),
        mesh=_mesh(),
        compiler_params=pltpu.CompilerParams(needs_layout_passes=False, use_tc_tiling_on_sc=False),
        scratch_types=[
            pltpu.VMEM((EPW,), jnp.int32),
            pltpu.VMEM((EPW,), jnp.int32),
            pltpu.VMEM((2 * NB, CH, D), jnp.float32),
            pltpu.VMEM_SHARED((N, D), jnp.float32),
            pltpu.SemaphoreType.DMA,
            pltpu.SemaphoreType.DMA,
            pltpu.SemaphoreType.DMA,
        ],
    )
    def k(hwp_hbm, ei_hbm, out_hbm, sidx_v, didx_v, rows_v, acc_sh,
          gsem, ssem0, ssem1):
        c = lax.axis_index("c")
        s = lax.axis_index("s")
        wid = s * NC + c
        zeros = jnp.zeros((16,), jnp.float32)

        ebase = wid * EPW
        icp0 = pltpu.async_copy(ei_hbm.at[0].at[pl.ds(ebase, EPW)], sidx_v, gsem)
        icp1 = pltpu.async_copy(ei_hbm.at[1].at[pl.ds(ebase, EPW)], didx_v, gsem)

        def zero_body(i, carry):
            for j in range(D // 16):
                rows_v[0, i, pl.ds(j * 16, 16)] = zeros
            return carry

        lax.fori_loop(0, CH, zero_body, 0)
        for t in range(5):
            cid = s + NS * t
            base = pl.multiple_of(cid * CH, CH)

            @pl.when(cid < N // CH)
            def _():
                pltpu.sync_copy(rows_v.at[0], acc_sh.at[pl.ds(base, CH)])

            @pl.when(cid == N // CH)
            def _():
                pltpu.sync_copy(rows_v.at[0].at[pl.ds(0, N % CH)],
                                acc_sh.at[pl.ds((N // CH) * CH, N % CH)])

        icp0.wait()
        icp1.wait()
        plsc.subcore_barrier()

        dummy = hwp_hbm.at[pl.ds(0, CH)]

        def fire_gathers(g, s0, nb=NB):
            gb = g * (NB * CH)
            for b in range(nb):
                pltpu.async_copy(
                    hwp_hbm.at[sidx_v.at[pl.ds(gb + b * CH, CH)]],
                    rows_v.at[s0 + b], gsem)

        def drain(sem_, s0, nb=NB):
            for b in range(nb):
                pltpu.make_async_copy(dummy, rows_v.at[s0 + b], sem_).wait()

        def fire_adds(g, s0, sem_, nb=NB):
            gb = g * (NB * CH)
            for b in range(nb):
                pltpu.async_copy(
                    rows_v.at[s0 + b],
                    acc_sh.at[didx_v.at[pl.ds(gb + b * CH, CH)]],
                    sem_, add=True)

        fire_gathers(0, 0)

        def p_body(kk, carry):
            g0 = 2 * kk
            drain(gsem, 0)
            fire_adds(g0, 0, ssem0)

            @pl.when(kk > 0)
            def _():
                drain(ssem1, NB)

            fire_gathers(g0 + 1, NB)
            drain(gsem, NB)
            fire_adds(g0 + 1, NB, ssem1)
            drain(ssem0, 0)

            @pl.when(kk < NPAIR - 1)
            def _():
                fire_gathers(g0 + 2, 0)

            return carry

        lax.fori_loop(0, NPAIR, p_body, 0)
        ge = 2 * NPAIR
        fire_gathers(ge, 0)
        drain(gsem, 0)
        fire_adds(ge, 0, ssem0)
        drain(ssem1, NB)
        fire_gathers(ge + 1, NB, NREM - NB)
        drain(gsem, NB, NREM - NB)
        fire_adds(ge + 1, NB, ssem1, NREM - NB)
        drain(ssem0, 0)
        drain(ssem1, NB, NREM - NB)

        tb = NFULL * CH
        pltpu.async_copy(hwp_hbm.at[sidx_v.at[pl.ds(tb, TAIL)]],
                         rows_v.at[0].at[pl.ds(0, TAIL)], gsem).wait()
        pltpu.sync_copy(rows_v.at[0].at[pl.ds(0, TAIL)],
                        acc_sh.at[didx_v.at[pl.ds(tb, TAIL)]], add=True)

        plsc.subcore_barrier()

        for t in range(5):
            cid = s + NS * t
            base = pl.multiple_of(cid * CH, CH)

            @pl.when(cid < N // CH)
            def _():
                pltpu.sync_copy(acc_sh.at[pl.ds(base, CH)],
                                out_hbm.at[pl.ds(c * N + base, CH)])

            @pl.when(cid == N // CH)
            def _():
                tail = N % CH
                t0 = (N // CH) * CH
                pltpu.sync_copy(acc_sh.at[pl.ds(t0, tail)],
                                out_hbm.at[pl.ds(c * N + t0, tail)])

    return k(hwp, edge_index)


def _lrelu(v):
    return jnp.where(v >= 0, v, 0.01 * v)


def _dinv_of(pdeg_blk):
    deg = jnp.sum(pdeg_blk, axis=0) + 1.0
    return lax.rsqrt(deg)[:, None]


def _dot(a, b):
    return jnp.dot(a, b, preferred_element_type=jnp.float32)


def _tc_encoder(x, W1, b1, W2, b2, g1W, pdeg):
    def body(x_r, w1_r, b1_r, w2_r, b2_r, g1w_r, pdeg_r, out_r, dinv_r):
        h = _lrelu(_dot(x_r[...], w1_r[...]) + b1_r[...])
        h = _lrelu(_dot(h, w2_r[...]) + b2_r[...])
        dinv = _dinv_of(pdeg_r[...])
        out_r[...] = _dot(h, g1w_r[...]) * dinv
        dinv_r[...] = dinv

    full = lambda shape: pl.BlockSpec(shape, lambda i: (0,) * len(shape))
    return pl.pallas_call(
        body,
        grid=(pl.cdiv(N, BNE),),
        in_specs=[
            pl.BlockSpec((BNE, 128), lambda i: (i, 0)),
            full((128, 128)),
            full((1, 128)),
            full((128, D)),
            full((1, D)),
            full((D, D)),
            pl.BlockSpec((NW, BNE), lambda i: (0, i)),
        ],
        out_specs=[pl.BlockSpec((BNE, D), lambda i: (i, 0)),
                   pl.BlockSpec((BNE, 1), lambda i: (i, 0))],
        out_shape=[jax.ShapeDtypeStruct((N, D), jnp.float32),
                   jax.ShapeDtypeStruct((N, 1), jnp.float32)],
    )(x, W1, b1, W2, b2, g1W, pdeg)


def _tc_combine(part, hwp, dinv, gW, gb):
    def body(p0_r, p1_r, hwp_r, dinv_r, gw_r, gb_r, out_r):
        dinv = dinv_r[...]
        h = _lrelu((p0_r[...] + p1_r[...] + hwp_r[...]) * dinv + gb_r[...])
        out_r[...] = _dot(h, gw_r[...]) * dinv

    full = lambda shape: pl.BlockSpec(shape, lambda i: (0,) * len(shape))
    blk = pl.BlockSpec((BN, D), lambda i: (i, 0))
    return pl.pallas_call(
        body,
        grid=(NBLK,),
        in_specs=[pl.BlockSpec((BN, D), lambda i: (i, 0)),
                  pl.BlockSpec((BN, D), lambda i: (i + NBLK, 0)),
                  blk,
                  pl.BlockSpec((BN, 1), lambda i: (i, 0)),
                  full((D, D)), full((1, D))],
        out_specs=pl.BlockSpec((BN, D), lambda i: (i, 0)),
        out_shape=jax.ShapeDtypeStruct((N, D), jnp.float32),
    )(part, part, hwp, dinv, gW, gb)


def _tc_final(part, hwp, dinv, gb, cW, cb):
    def body(p0_r, p1_r, hwp_r, dinv_r, gb_r, cw_r, cb_r, h_r, log_r):
        dinv = dinv_r[...]
        h = _lrelu((p0_r[...] + p1_r[...] + hwp_r[...]) * dinv + gb_r[...])
        h_r[...] = h
        log_r[...] = _dot(h, cw_r[...]) + cb_r[...]

    full = lambda shape: pl.BlockSpec(shape, lambda i: (0,) * len(shape))
    return pl.pallas_call(
        body,
        grid=(NBLK,),
        in_specs=[pl.BlockSpec((BN, D), lambda i: (i, 0)),
                  pl.BlockSpec((BN, D), lambda i: (i + NBLK, 0)),
                  pl.BlockSpec((BN, D), lambda i: (i, 0)),
                  pl.BlockSpec((BN, 1), lambda i: (i, 0)),
                  full((1, D)), full((D, 2)), full((1, 2))],
        out_specs=[pl.BlockSpec((BN, D), lambda i: (i, 0)),
                   pl.BlockSpec((BN, 2), lambda i: (i, 0))],
        out_shape=[jax.ShapeDtypeStruct((N, D), jnp.float32),
                   jax.ShapeDtypeStruct((N, 2), jnp.float32)],
    )(part, part, hwp, dinv, gb, cW, cb)


def kernel(x, edge_index, W1, b1, W2, b2, g1W, g1b, g2W, g2b, cW, cb):
    pdeg = _sc_deg(edge_index)
    hw1p, dinv = _tc_encoder(x, W1, b1.reshape(1, -1), W2, b2.reshape(1, -1),
                             g1W, pdeg)
    part1 = _sc_scatter(hw1p, edge_index)
    hw2p = _tc_combine(part1, hw1p, dinv, g2W, g1b.reshape(1, -1))
    part2 = _sc_scatter(hw2p, edge_index)
    h, logits = _tc_final(part2, hw2p, dinv, g2b.reshape(1, -1),
                          cW, cb.reshape(1, -1))
    return (logits, h)


# speedup vs baseline: 46.3884x; 1.0050x over previous
import functools
import jax
import jax.numpy as jnp
from jax import lax
from jax.experimental import pallas as pl
from jax.experimental.pallas import tpu as pltpu
from jax.experimental.pallas import tpu_sc as plsc

N = 10000
E = 320000
D = 64
NC = 2
NS = 16
NW = NC * NS
CH = 128
EPW = E // NW
NFULL = EPW // CH
TAIL = EPW - NFULL * CH
NB = 4
NPAIR = NFULL // (2 * NB)
NREM = NFULL - 2 * NPAIR * NB

BNE = 2048
BN = 2000
NBLK = N // BN

_mesh = lambda: plsc.VectorSubcoreMesh(core_axis_name="c", subcore_axis_name="s")


def _sc_deg(edge_index):
    @functools.partial(
        pl.kernel,
        out_type=jax.ShapeDtypeStruct((NW, N), jnp.float32),
        mesh=_mesh(),
        compiler_params=pltpu.CompilerParams(needs_layout_passes=False, use_tc_tiling_on_sc=False),
        scratch_types=[
            pltpu.VMEM((N,), jnp.float32),
            pltpu.VMEM((EPW,), jnp.int32),
            pltpu.SemaphoreType.DMA,
        ],
    )
    def k(ei_hbm, pdeg_hbm, deg_v, idx_v, sem):
        c = lax.axis_index("c")
        s = lax.axis_index("s")
        wid = s * NC + c
        zeros = jnp.zeros((16,), jnp.float32)
        ones = jnp.ones((16,), jnp.float32)

        icp = pltpu.async_copy(ei_hbm.at[1].at[pl.ds(wid * EPW, EPW)], idx_v, sem)

        def zero_body(i, carry):
            deg_v[pl.ds(i * 16, 16)] = zeros
            return carry

        lax.fori_loop(0, N // 16, zero_body, 0)
        icp.wait()

        def h_body(i, carry):
            for j in range(25):
                idx = idx_v[pl.ds((i * 25 + j) * 16, 16)]
                plsc.addupdate_scatter(deg_v, [idx], ones)
            return carry

        lax.fori_loop(0, EPW // (25 * 16), h_body, 0)

        pltpu.sync_copy(deg_v, pdeg_hbm.at[wid])

    return k(edge_index)


def _sc_scatter(hwp, edge_index):
    @functools.partial(
        pl.kernel,
        out_type=jax.ShapeDtypeStruct((NC * N, D), jnp.float32),
        mesh=_mesh(),
        compiler_params=pltpu.CompilerParams(needs_layout_passes=False, use_tc_tiling_on_sc=False),
        scratch_types=[
            pltpu.VMEM((EPW,), jnp.int32),
            pltpu.VMEM((EPW,), jnp.int32),
            pltpu.VMEM((2 * NB, CH, D), jnp.float32),
            pltpu.VMEM_SHARED((N, D), jnp.float32),
            pltpu.SemaphoreType.DMA,
            pltpu.SemaphoreType.DMA,
            pltpu.SemaphoreType.DMA,
        ],
    )
    def k(hwp_hbm, ei_hbm, out_hbm, sidx_v, didx_v, rows_v, acc_sh,
          gsem, ssem0, ssem1):
        c = lax.axis_index("c")
        s = lax.axis_index("s")
        wid = s * NC + c
        zeros = jnp.zeros((16,), jnp.float32)

        ebase = wid * EPW
        icp0 = pltpu.async_copy(ei_hbm.at[0].at[pl.ds(ebase, EPW)], sidx_v, gsem)
        icp1 = pltpu.async_copy(ei_hbm.at[1].at[pl.ds(ebase, EPW)], didx_v, gsem)

        def zero_body(i, carry):
            for j in range(D // 16):
                rows_v[0, i, pl.ds(j * 16, 16)] = zeros
            return carry

        lax.fori_loop(0, CH, zero_body, 0)
        for t in range(5):
            cid = s + NS * t
            base = pl.multiple_of(cid * CH, CH)

            @pl.when(cid < N // CH)
            def _():
                pltpu.sync_copy(rows_v.at[0], acc_sh.at[pl.ds(base, CH)])

            @pl.when(cid == N // CH)
            def _():
                pltpu.sync_copy(rows_v.at[0].at[pl.ds(0, N % CH)],
                                acc_sh.at[pl.ds((N // CH) * CH, N % CH)])

        icp0.wait()
        icp1.wait()
        plsc.subcore_barrier()

        dummy = hwp_hbm.at[pl.ds(0, CH)]

        def fire_gathers(g, s0, nb=NB):
            gb = g * (NB * CH)
            for b in range(nb):
                pltpu.async_copy(
                    hwp_hbm.at[sidx_v.at[pl.ds(gb + b * CH, CH)]],
                    rows_v.at[s0 + b], gsem)

        def drain(sem_, s0, nb=NB):
            for b in range(nb):
                pltpu.make_async_copy(dummy, rows_v.at[s0 + b], sem_).wait()

        def fire_adds(g, s0, sem_, nb=NB):
            gb = g * (NB * CH)
            for b in range(nb):
                pltpu.async_copy(
                    rows_v.at[s0 + b],
                    acc_sh.at[didx_v.at[pl.ds(gb + b * CH, CH)]],
                    sem_, add=True)

        fire_gathers(0, 0)

        def p_body(kk, carry):
            g0 = 2 * kk
            drain(gsem, 0)
            fire_adds(g0, 0, ssem0)

            @pl.when(kk > 0)
            def _():
                drain(ssem1, NB)

            fire_gathers(g0 + 1, NB)
            drain(gsem, NB)
            fire_adds(g0 + 1, NB, ssem1)
            drain(ssem0, 0)

            @pl.when(kk < NPAIR - 1)
            def _():
                fire_gathers(g0 + 2, 0)

            return carry

        lax.fori_loop(0, NPAIR, p_body, 0)
        ge = 2 * NPAIR
        fire_gathers(ge, 0)
        drain(gsem, 0)
        fire_adds(ge, 0, ssem0)
        drain(ssem1, NB)
        fire_gathers(ge + 1, NB, NREM - NB)
        drain(gsem, NB, NREM - NB)
        fire_adds(ge + 1, NB, ssem1, NREM - NB)
        drain(ssem0, 0)
        drain(ssem1, NB, NREM - NB)

        tb = NFULL * CH
        pltpu.async_copy(hwp_hbm.at[sidx_v.at[pl.ds(tb, TAIL)]],
                         rows_v.at[0].at[pl.ds(0, TAIL)], gsem).wait()
        pltpu.sync_copy(rows_v.at[0].at[pl.ds(0, TAIL)],
                        acc_sh.at[didx_v.at[pl.ds(tb, TAIL)]], add=True)

        plsc.subcore_barrier()

        for t in range(5):
            cid = s + NS * t
            base = pl.multiple_of(cid * CH, CH)

            @pl.when(cid < N // CH)
            def _():
                pltpu.sync_copy(acc_sh.at[pl.ds(base, CH)],
                                out_hbm.at[pl.ds(c * N + base, CH)])

            @pl.when(cid == N // CH)
            def _():
                tail = N % CH
                t0 = (N // CH) * CH
                pltpu.sync_copy(acc_sh.at[pl.ds(t0, tail)],
                                out_hbm.at[pl.ds(c * N + t0, tail)])

    return k(hwp, edge_index)


def _lrelu(v):
    return jnp.where(v >= 0, v, 0.01 * v)


def _dinv_of(pdeg_blk):
    deg = jnp.sum(pdeg_blk, axis=0) + 1.0
    return lax.rsqrt(deg)[:, None]


def _dot(a, b):
    return jnp.dot(a, b, preferred_element_type=jnp.float32)


def _tc_encoder(x, W1, b1, W2, b2, g1W, pdeg):
    def body(x_r, w1_r, b1_r, w2_r, b2_r, g1w_r, pdeg_r, out_r, dinv_r):
        h = _lrelu(_dot(x_r[...], w1_r[...]) + b1_r[...])
        h = _lrelu(_dot(h, w2_r[...]) + b2_r[...])
        dinv = _dinv_of(pdeg_r[...])
        out_r[...] = _dot(h, g1w_r[...]) * dinv
        dinv_r[...] = dinv

    full = lambda shape: pl.BlockSpec(shape, lambda i: (0,) * len(shape))
    return pl.pallas_call(
        body,
        grid=(pl.cdiv(N, BNE),),
        in_specs=[
            pl.BlockSpec((BNE, 128), lambda i: (i, 0)),
            full((128, 128)),
            full((1, 128)),
            full((128, D)),
            full((1, D)),
            full((D, D)),
            pl.BlockSpec((NW, BNE), lambda i: (0, i)),
        ],
        out_specs=[pl.BlockSpec((BNE, D), lambda i: (i, 0)),
                   pl.BlockSpec((BNE, 1), lambda i: (i, 0))],
        out_shape=[jax.ShapeDtypeStruct((N, D), jnp.float32),
                   jax.ShapeDtypeStruct((N, 1), jnp.float32)],
    )(x, W1, b1, W2, b2, g1W, pdeg)


def _tc_combine(part, hwp, dinv, gW, gb):
    def body(p0_r, p1_r, hwp_r, dinv_r, gw_r, gb_r, out_r):
        dinv = dinv_r[...]
        h = _lrelu((p0_r[...] + p1_r[...] + hwp_r[...]) * dinv + gb_r[...])
        out_r[...] = _dot(h, gw_r[...]) * dinv

    full = lambda shape: pl.BlockSpec(shape, lambda i: (0,) * len(shape))
    blk = pl.BlockSpec((BN, D), lambda i: (i, 0))
    return pl.pallas_call(
        body,
        grid=(NBLK,),
        in_specs=[pl.BlockSpec((BN, D), lambda i: (i, 0)),
                  pl.BlockSpec((BN, D), lambda i: (i + NBLK, 0)),
                  blk,
                  pl.BlockSpec((BN, 1), lambda i: (i, 0)),
                  full((D, D)), full((1, D))],
        out_specs=pl.BlockSpec((BN, D), lambda i: (i, 0)),
        out_shape=jax.ShapeDtypeStruct((N, D), jnp.float32),
    )(part, part, hwp, dinv, gW, gb)


def _tc_final(part, hwp, dinv, gb, cW, cb):
    def body(p0_r, p1_r, hwp_r, dinv_r, gb_r, cw_r, cb_r, h_r, log_r):
        dinv = dinv_r[...]
        h = _lrelu((p0_r[...] + p1_r[...] + hwp_r[...]) * dinv + gb_r[...])
        h_r[...] = h
        log_r[...] = _dot(h, cw_r[...]) + cb_r[...]

    full = lambda shape: pl.BlockSpec(shape, lambda i: (0,) * len(shape))
    return pl.pallas_call(
        body,
        grid=(NBLK,),
        in_specs=[pl.BlockSpec((BN, D), lambda i: (i, 0)),
                  pl.BlockSpec((BN, D), lambda i: (i + NBLK, 0)),
                  pl.BlockSpec((BN, D), lambda i: (i, 0)),
                  pl.BlockSpec((BN, 1), lambda i: (i, 0)),
                  full((1, D)), full((D, 2)), full((1, 2))],
        out_specs=[pl.BlockSpec((BN, D), lambda i: (i, 0)),
                   pl.BlockSpec((BN, 2), lambda i: (i, 0))],
        out_shape=[jax.ShapeDtypeStruct((N, D), jnp.float32),
                   jax.ShapeDtypeStruct((N, 2), jnp.float32)],
    )(part, part, hwp, dinv, gb, cW, cb)


def kernel(x, edge_index, W1, b1, W2, b2, g1W, g1b, g2W, g2b, cW, cb):
    pdeg = _sc_deg(edge_index)
    hw1p, dinv = _tc_encoder(x, W1, b1.reshape(1, -1), W2, b2.reshape(1, -1),
                             g1W, pdeg)
    part1 = _sc_scatter(hw1p, edge_index)
    hw2p = _tc_combine(part1, hw1p, dinv, g2W, g1b.reshape(1, -1))
    part2 = _sc_scatter(hw2p, edge_index)
    h, logits = _tc_final(part2, hw2p, dinv, g2b.reshape(1, -1),
                          cW, cb.reshape(1, -1))
    return (logits, h)
